# baseline (speedup 1.0000x reference)
import jax
import jax.numpy as jnp
from jax import lax
from jax.experimental import pallas as pl
from jax.experimental.pallas import tpu as pltpu
from jax.experimental.pallas import tpu_sc as plsc

_MASK = 1_000_000
_TRASH = 65_536
_OUT = _MASK + _TRASH
_HALF = _MASK // 2
_NIDS = 100_000
_NSUB = 16
_NCORE = 2
_PAD_TO = 102_400
_PER_W = _PAD_TO // _NSUB
_CH = 128
_N_CH = _PER_W // _CH
_SLICE = 31_248
_TAIL = _HALF - _NSUB * _SLICE
_ZB = _SLICE // 3


def _scatter_body(ids_hbm, out_hbm, idx_v, sidx, ones_v, zbuf, sem):
    c = lax.axis_index("c")
    s = lax.axis_index("s")
    base = c * _HALF

    idx_dma = pltpu.async_copy(ids_hbm.at[pl.ds(s * _PER_W, _PER_W)], idx_v, sem)

    zvec = jnp.zeros((16,), jnp.int32)
    ovec = jnp.ones((16,), jnp.int32)
    for k in range(_CH // 16):
        ones_v[pl.ds(k * 16, 16)] = ovec

    @pl.loop(0, _ZB // 16)
    def _(i):
        zbuf[pl.ds(i * 16, 16)] = zvec

    for k in range(_SLICE // _ZB):
        pltpu.sync_copy(zbuf, out_hbm.at[pl.ds(base + s * _SLICE + k * _ZB, _ZB)])

    @pl.when(s == _NSUB - 1)
    def _():
        pltpu.sync_copy(zbuf.at[pl.ds(0, _TAIL)],
                        out_hbm.at[pl.ds(base + _NSUB * _SLICE, _TAIL)])

    idx_dma.wait()
    plsc.subcore_barrier()

    @pl.loop(0, _N_CH)
    def _(jc):
        for k in range(_CH // 16):
            v = idx_v[pl.ds(jc * _CH + k * 16, 16)]
            inr = (v >= base) & (v < base + _HALF)
            safe = jnp.where(inr, v, _MASK + (v & (_TRASH - 1)))
            sidx[jc, pl.ds(k * 16, 16)] = safe

    @pl.loop(0, _N_CH)
    def _(jc):
        pltpu.sync_copy(ones_v, out_hbm.at[sidx.at[jc]])


def kernel(in_ids, size_tensor):
    assert size_tensor.shape[0] == _MASK and in_ids.shape[0] == _NIDS
    ids = in_ids.astype(jnp.int32)
    pad = -1 - jnp.arange(_PAD_TO - _NIDS, dtype=jnp.int32)
    ids = jnp.concatenate([ids, pad])

    mesh = plsc.VectorSubcoreMesh(core_axis_name="c", subcore_axis_name="s",
                                  num_cores=_NCORE, num_subcores=_NSUB)
    run = pl.kernel(
        _scatter_body,
        out_type=jax.ShapeDtypeStruct((_OUT,), jnp.int32),
        mesh=mesh,
        compiler_params=pltpu.CompilerParams(needs_layout_passes=False),
        scratch_types=[
            pltpu.VMEM((_PER_W,), jnp.int32),
            pltpu.VMEM((_N_CH, _CH), jnp.int32),
            pltpu.VMEM((_CH,), jnp.int32),
            pltpu.VMEM((_ZB,), jnp.int32),
            pltpu.SemaphoreType.DMA,
        ],
    )
    return run(ids)[:_MASK].astype(jnp.bool_)


# speedup vs baseline: 1.6463x; 1.6463x over previous
import jax
import jax.numpy as jnp
from jax import lax
from jax.experimental import pallas as pl
from jax.experimental.pallas import tpu as pltpu
from jax.experimental.pallas import tpu_sc as plsc

_MASK = 1_000_000
_HALF = _MASK // 2
_NIDS = 100_000
_NSUB = 16
_NCORE = 2
_PAD_TO = 102_400
_PER_W = _PAD_TO // _NSUB
_CH = 128
_N_CH = _PER_W // _CH
_HW = _HALF // 4
_WSL = 7_808
_WTAIL = _HW - _NSUB * _WSL


def _scatter_body(ids_hbm, out_hbm, half, idx_v, sidx, sval, zbuf, stage_v, sem):
    c = lax.axis_index("c")
    s = lax.axis_index("s")
    base = c * _HALF

    idx_dma = pltpu.async_copy(ids_hbm.at[pl.ds(s * _PER_W, _PER_W)], idx_v, sem)

    zvec = jnp.zeros((16,), jnp.int32)

    @pl.loop(0, _WSL // 16)
    def _(i):
        zbuf[pl.ds(i * 16, 16)] = zvec

    pltpu.sync_copy(zbuf, half.at[pl.ds(s * _WSL, _WSL)])

    @pl.when(s == _NSUB - 1)
    def _():
        pltpu.sync_copy(zbuf.at[pl.ds(0, _WTAIL)],
                        half.at[pl.ds(_NSUB * _WSL, _WTAIL)])

    idx_dma.wait()
    plsc.subcore_barrier()

    @pl.loop(0, _N_CH)
    def _(jc):
        for k in range(_CH // 16):
            v = idx_v[pl.ds(jc * _CH + k * 16, 16)]
            local = v - base
            inr = (local >= 0) & (local < _HALF)
            w = jnp.where(inr, local >> 2, (v & 0x3FFFF) >> 2)
            val = jnp.where(inr, 1 << ((local & 3) << 3), 0)
            sidx[jc, pl.ds(k * 16, 16)] = w
            sval[jc, pl.ds(k * 16, 16)] = val

    @pl.loop(0, _N_CH)
    def _(jc):
        pltpu.sync_copy(sval.at[jc], half.at[sidx.at[jc]], add=True)

    plsc.subcore_barrier()

    pltpu.sync_copy(half.at[pl.ds(s * _WSL, _WSL)], stage_v)
    pltpu.sync_copy(stage_v, out_hbm.at[pl.ds(c * _HW + s * _WSL, _WSL)])

    @pl.when(s == _NSUB - 1)
    def _():
        pltpu.sync_copy(half.at[pl.ds(_NSUB * _WSL, _WTAIL)],
                        stage_v.at[pl.ds(0, _WTAIL)])
        pltpu.sync_copy(stage_v.at[pl.ds(0, _WTAIL)],
                        out_hbm.at[pl.ds(c * _HW + _NSUB * _WSL, _WTAIL)])


def kernel(in_ids, size_tensor):
    assert size_tensor.shape[0] == _MASK and in_ids.shape[0] == _NIDS
    ids = in_ids.astype(jnp.int32)
    pad = -1 - jnp.arange(_PAD_TO - _NIDS, dtype=jnp.int32)
    ids = jnp.concatenate([ids, pad])

    mesh = plsc.VectorSubcoreMesh(core_axis_name="c", subcore_axis_name="s",
                                  num_cores=_NCORE, num_subcores=_NSUB)
    run = pl.kernel(
        _scatter_body,
        out_type=jax.ShapeDtypeStruct((_MASK // 4,), jnp.int32),
        mesh=mesh,
        compiler_params=pltpu.CompilerParams(needs_layout_passes=False),
        scratch_types=[
            pltpu.VMEM_SHARED((_HW,), jnp.int32),
            pltpu.VMEM((_PER_W,), jnp.int32),
            pltpu.VMEM((_N_CH, _CH), jnp.int32),
            pltpu.VMEM((_N_CH, _CH), jnp.int32),
            pltpu.VMEM((_WSL,), jnp.int32),
            pltpu.VMEM((_WSL,), jnp.int32),
            pltpu.SemaphoreType.DMA,
        ],
    )
    return run(ids).view(jnp.uint8).astype(jnp.bool_)


# speedup vs baseline: 4.8260x; 2.9314x over previous
import jax
import jax.numpy as jnp
from jax import lax
from jax.experimental import pallas as pl
from jax.experimental.pallas import tpu as pltpu
from jax.experimental.pallas import tpu_sc as plsc

_MASK = 1_000_000
_NIDS = 100_000
_NSUB = 16
_NCORE = 2
_PAD_TO = 102_400
_PER_W = _PAD_TO // _NSUB
_CH = 128
_N_CH = _PER_W // _CH
_NW = _MASK // 4
_HW = _NW // 2
_WSL = 7_808
_WTAIL = _HW - _NSUB * _WSL


def _scatter_body(ids_hbm, out_hbm, half, idx_v, sidx, sval, zbuf, stage_v, sem):
    c = lax.axis_index("c")
    s = lax.axis_index("s")
    wbase = c * _HW

    idx_dma = pltpu.async_copy(ids_hbm.at[pl.ds(s * _PER_W, _PER_W)], idx_v, sem)

    zvec = jnp.zeros((16,), jnp.int32)

    @pl.loop(0, _WSL // 16)
    def _(i):
        zbuf[pl.ds(i * 16, 16)] = zvec

    pltpu.sync_copy(zbuf, half.at[pl.ds(s * _WSL, _WSL)])

    @pl.when(s == _NSUB - 1)
    def _():
        pltpu.sync_copy(zbuf.at[pl.ds(0, _WTAIL)],
                        half.at[pl.ds(_NSUB * _WSL, _WTAIL)])

    idx_dma.wait()
    plsc.subcore_barrier()

    @pl.loop(0, _N_CH)
    def _(jc):
        for k in range(_CH // 16):
            v = idx_v[pl.ds(jc * _CH + k * 16, 16)]
            ge2 = v >= 2 * _NW
            t = jnp.where(ge2, v - 2 * _NW, v)
            ge1 = t >= _NW
            w = jnp.where(ge1, t - _NW, t)
            plane = (ge2.astype(jnp.int32) << 1) | ge1.astype(jnp.int32)
            inr = (w >= wbase) & (w < wbase + _HW)
            widx = jnp.where(inr, w - wbase, (v & 0x3FFFF) >> 2)
            val = jnp.where(inr, 1 << (plane << 3), 0)
            sidx[jc, pl.ds(k * 16, 16)] = widx
            sval[jc, pl.ds(k * 16, 16)] = val

    @pl.loop(0, _N_CH)
    def _(jc):
        pltpu.sync_copy(sval.at[jc], half.at[sidx.at[jc]], add=True)

    plsc.subcore_barrier()

    pltpu.sync_copy(half.at[pl.ds(s * _WSL, _WSL)], stage_v)
    pltpu.sync_copy(stage_v, out_hbm.at[pl.ds(wbase + s * _WSL, _WSL)])

    @pl.when(s == _NSUB - 1)
    def _():
        pltpu.sync_copy(half.at[pl.ds(_NSUB * _WSL, _WTAIL)],
                        stage_v.at[pl.ds(0, _WTAIL)])
        pltpu.sync_copy(stage_v.at[pl.ds(0, _WTAIL)],
                        out_hbm.at[pl.ds(wbase + _NSUB * _WSL, _WTAIL)])


def kernel(in_ids, size_tensor):
    assert size_tensor.shape[0] == _MASK and in_ids.shape[0] == _NIDS
    ids = in_ids.astype(jnp.int32)
    pad = -1 - jnp.arange(_PAD_TO - _NIDS, dtype=jnp.int32)
    ids = jnp.concatenate([ids, pad])

    mesh = plsc.VectorSubcoreMesh(core_axis_name="c", subcore_axis_name="s",
                                  num_cores=_NCORE, num_subcores=_NSUB)
    run = pl.kernel(
        _scatter_body,
        out_type=jax.ShapeDtypeStruct((_NW,), jnp.int32),
        mesh=mesh,
        compiler_params=pltpu.CompilerParams(needs_layout_passes=False),
        scratch_types=[
            pltpu.VMEM_SHARED((_HW,), jnp.int32),
            pltpu.VMEM((_PER_W,), jnp.int32),
            pltpu.VMEM((_N_CH, _CH), jnp.int32),
            pltpu.VMEM((_N_CH, _CH), jnp.int32),
            pltpu.VMEM((_WSL,), jnp.int32),
            pltpu.VMEM((_WSL,), jnp.int32),
            pltpu.SemaphoreType.DMA,
        ],
    )
    w = run(ids)
    return jnp.concatenate([(w >> 8 * p) & 0xFF for p in range(4)]) != 0


# speedup vs baseline: 9.6301x; 1.9955x over previous
import jax
import jax.numpy as jnp
from jax import lax
from jax.experimental import pallas as pl
from jax.experimental.pallas import tpu as pltpu
from jax.experimental.pallas import tpu_sc as plsc

_MASK = 1_000_000
_NIDS = 100_000
_NSUB = 16
_NCORE = 2
_PAD_TO = 102_400
_PER_W = _PAD_TO // _NSUB
_CH = 128
_N_CH = _PER_W // _CH
_NW = 1 << 18
_HW = _NW // 2
_WSL = _HW // _NSUB


def _scatter_body(ids_hbm, out_hbm, half, idx_v, sidx, sval, zbuf, stage_v, sem):
    c = lax.axis_index("c")
    s = lax.axis_index("s")
    wbase = c * _HW

    idx_dma = pltpu.async_copy(ids_hbm.at[pl.ds(s * _PER_W, _PER_W)], idx_v, sem)

    zvec = jnp.zeros((16,), jnp.int32)

    @pl.loop(0, _WSL // 16)
    def _(i):
        zbuf[pl.ds(i * 16, 16)] = zvec

    pltpu.sync_copy(zbuf, half.at[pl.ds(s * _WSL, _WSL)])

    idx_dma.wait()
    plsc.subcore_barrier()

    @pl.loop(0, _N_CH)
    def _(jc):
        for k in range(_CH // 16):
            v = idx_v[pl.ds(jc * _CH + k * 16, 16)]
            w = v & (_NW - 1)
            plane = (v >> 18) & 3
            inr = (v >= 0) & (w >= wbase) & (w < wbase + _HW)
            widx = jnp.where(inr, w - wbase, v & 0xFFFF)
            val = jnp.where(inr, 1 << (plane << 3), 0)
            sidx[jc, pl.ds(k * 16, 16)] = widx
            sval[jc, pl.ds(k * 16, 16)] = val

    @pl.loop(0, _N_CH)
    def _(jc):
        pltpu.sync_copy(sval.at[jc], half.at[sidx.at[jc]], add=True)

    plsc.subcore_barrier()

    pltpu.sync_copy(half.at[pl.ds(s * _WSL, _WSL)], stage_v)
    pltpu.sync_copy(stage_v, out_hbm.at[pl.ds(wbase + s * _WSL, _WSL)])


def kernel(in_ids, size_tensor):
    assert size_tensor.shape[0] == _MASK and in_ids.shape[0] == _NIDS
    ids = in_ids.astype(jnp.int32)
    pad = -1 - jnp.arange(_PAD_TO - _NIDS, dtype=jnp.int32)
    ids = jnp.concatenate([ids, pad])

    mesh = plsc.VectorSubcoreMesh(core_axis_name="c", subcore_axis_name="s",
                                  num_cores=_NCORE, num_subcores=_NSUB)
    run = pl.kernel(
        _scatter_body,
        out_type=jax.ShapeDtypeStruct((_NW,), jnp.int32),
        mesh=mesh,
        compiler_params=pltpu.CompilerParams(needs_layout_passes=False),
        scratch_types=[
            pltpu.VMEM_SHARED((_HW,), jnp.int32),
            pltpu.VMEM((_PER_W,), jnp.int32),
            pltpu.VMEM((_N_CH, _CH), jnp.int32),
            pltpu.VMEM((_N_CH, _CH), jnp.int32),
            pltpu.VMEM((_WSL,), jnp.int32),
            pltpu.VMEM((_WSL,), jnp.int32),
            pltpu.SemaphoreType.DMA,
        ],
    )
    w = run(ids)
    planes = [(w >> (8 * p)) & 0xFF for p in range(4)]
    planes[3] = planes[3][: _MASK - 3 * _NW]
    return jnp.concatenate(planes) != 0


# speedup vs baseline: 10.8812x; 1.1299x over previous
import jax
import jax.numpy as jnp
from jax import lax
from jax.experimental import pallas as pl
from jax.experimental.pallas import tpu as pltpu
from jax.experimental.pallas import tpu_sc as plsc

_MASK = 1_000_000
_NIDS = 100_000
_NSUB = 16
_NCORE = 2
_SHARE = _NIDS // _NSUB
_WIN = 6_272
_CH = 128
_N_CH = 49
_NW = 1 << 18
_HW = _NW // 2
_WSL = _HW // _NSUB
_WSL2 = _WSL // 2


def _scatter_body(ids_hbm, out_hbm, half, idx_v, sidx, sval, zbuf,
                  stage_a, stage_b, sem_i, sem_z, sem_s):
    c = lax.axis_index("c")
    s = lax.axis_index("s")
    wbase = c * _HW
    skew = (s * _SHARE) & 7
    start = pl.multiple_of(s * _SHARE - skew, 8)

    idx_dma = pltpu.async_copy(ids_hbm.at[pl.ds(start, _WIN - 16)],
                               idx_v.at[pl.ds(0, _WIN - 16)], sem_i)

    zvec = jnp.zeros((16,), jnp.int32)

    @pl.loop(0, _WSL // 16)
    def _(i):
        zbuf[pl.ds(i * 16, 16)] = zvec

    zero_dma = pltpu.async_copy(zbuf, half.at[pl.ds(s * _WSL, _WSL)], sem_z)

    idx_dma.wait()

    iota16 = lax.iota(jnp.int32, 16)

    def transform(jc, lo_mask, hi_mask):
        for k in range(_CH // 16):
            v = idx_v[pl.ds(jc * _CH + k * 16, 16)]
            w = v & (_NW - 1)
            inr = (v >= 0) & (((v >> 17) & 1) == c)
            if lo_mask or hi_mask:
                pos = jc * _CH + k * 16 + iota16
                if lo_mask:
                    inr &= pos >= skew
                if hi_mask:
                    inr &= pos < skew + _SHARE
            val = jnp.where(inr, 1 << (((v >> 18) & 3) << 3), 0)
            sidx[jc, pl.ds(k * 16, 16)] = v & (_HW - 1)
            sval[jc, pl.ds(k * 16, 16)] = val

    transform(0, True, False)

    @pl.loop(1, _N_CH - 1)
    def _(jc):
        transform(jc, False, False)

    transform(_N_CH - 1, False, True)

    zero_dma.wait()
    plsc.subcore_barrier()

    @pl.loop(0, _N_CH)
    def _(jc):
        pltpu.async_copy(sval.at[jc], half.at[sidx.at[jc]], sem_s, add=True)

    @pl.loop(0, _N_CH)
    def _(jc):
        pltpu.make_async_copy(sval.at[jc], half.at[sidx.at[jc]], sem_s).wait()

    plsc.subcore_barrier()

    d_a = pltpu.async_copy(half.at[pl.ds(s * _WSL, _WSL2)], stage_a, sem_z)
    d_b = pltpu.async_copy(half.at[pl.ds(s * _WSL + _WSL2, _WSL2)], stage_b,
                           sem_i)
    d_a.wait()
    o_a = pltpu.async_copy(stage_a, out_hbm.at[pl.ds(wbase + s * _WSL, _WSL2)],
                           sem_z)
    d_b.wait()
    o_b = pltpu.async_copy(stage_b,
                           out_hbm.at[pl.ds(wbase + s * _WSL + _WSL2, _WSL2)],
                           sem_i)
    o_a.wait()
    o_b.wait()


def kernel(in_ids, size_tensor):
    assert size_tensor.shape[0] == _MASK and in_ids.shape[0] == _NIDS
    ids = in_ids.astype(jnp.int32)

    mesh = plsc.VectorSubcoreMesh(core_axis_name="c", subcore_axis_name="s",
                                  num_cores=_NCORE, num_subcores=_NSUB)
    run = pl.kernel(
        _scatter_body,
        out_type=jax.ShapeDtypeStruct((_NW,), jnp.int32),
        mesh=mesh,
        compiler_params=pltpu.CompilerParams(needs_layout_passes=False),
        scratch_types=[
            pltpu.VMEM_SHARED((_HW,), jnp.int32),
            pltpu.VMEM((_WIN,), jnp.int32),
            pltpu.VMEM((_N_CH, _CH), jnp.int32),
            pltpu.VMEM((_N_CH, _CH), jnp.int32),
            pltpu.VMEM((_WSL,), jnp.int32),
            pltpu.VMEM((_WSL2,), jnp.int32),
            pltpu.VMEM((_WSL2,), jnp.int32),
            pltpu.SemaphoreType.DMA,
            pltpu.SemaphoreType.DMA,
            pltpu.SemaphoreType.DMA,
        ],
    )
    w = run(ids)
    planes = [(w >> (8 * p)) & 0xFF for p in range(4)]
    planes[3] = planes[3][: _MASK - 3 * _NW]
    return jnp.concatenate(planes) != 0


# speedup vs baseline: 10.9195x; 1.0035x over previous
import jax
import jax.numpy as jnp
from jax import lax
from jax.experimental import pallas as pl
from jax.experimental.pallas import tpu as pltpu
from jax.experimental.pallas import tpu_sc as plsc

_MASK = 1_000_000
_NIDS = 100_000
_NSUB = 16
_NCORE = 2
_SHARE = _NIDS // _NSUB
_WIN = 6_272
_CH = 128
_N_CH = 49
_NW = 1 << 18
_HW = _NW // 2
_WSL = _HW // _NSUB
_WSL2 = _WSL // 2


def _scatter_body(ids_hbm, out_hbm, half, idx_v, sidx, sval, zbuf,
                  stage_a, stage_b, sem_i, sem_z, sem_s):
    c = lax.axis_index("c")
    s = lax.axis_index("s")
    wbase = c * _HW
    skew = (s * _SHARE) & 7
    start = pl.multiple_of(s * _SHARE - skew, 8)

    idx_dma = pltpu.async_copy(ids_hbm.at[pl.ds(start, _WIN - 16)],
                               idx_v.at[pl.ds(0, _WIN - 16)], sem_i)

    zvec = jnp.zeros((16,), jnp.int32)

    @pl.loop(0, _WSL // 16)
    def _(i):
        zbuf[pl.ds(i * 16, 16)] = zvec

    zero_dma = pltpu.async_copy(zbuf, half.at[pl.ds(s * _WSL, _WSL)], sem_z)

    idx_dma.wait()

    iota16 = lax.iota(jnp.int32, 16)

    def transform(jc, lo_mask, hi_mask):
        for k in range(_CH // 16):
            v = idx_v[pl.ds(jc * _CH + k * 16, 16)]
            w = v & (_NW - 1)
            inr = (v >= 0) & (((v >> 17) & 1) == c)
            if lo_mask or hi_mask:
                pos = jc * _CH + k * 16 + iota16
                if lo_mask:
                    inr &= pos >= skew
                if hi_mask:
                    inr &= pos < skew + _SHARE
            val = jnp.where(inr, 1 << (((v >> 18) & 3) << 3), 0)
            sidx[jc, pl.ds(k * 16, 16)] = v & (_HW - 1)
            sval[jc, pl.ds(k * 16, 16)] = val

    transform(0, True, False)

    @pl.loop(1, _N_CH - 1)
    def _(jc):
        transform(jc, False, False)

    transform(_N_CH - 1, False, True)

    zero_dma.wait()
    plsc.subcore_barrier()

    @pl.loop(0, _N_CH)
    def _(jc):
        pltpu.async_copy(sval.at[jc], half.at[sidx.at[jc]], sem_s, add=True)

    @pl.loop(0, _N_CH)
    def _(jc):
        pltpu.make_async_copy(sval.at[jc], half.at[sidx.at[jc]], sem_s).wait()

    plsc.subcore_barrier()

    d_a = pltpu.async_copy(half.at[pl.ds(s * _WSL, _WSL2)], stage_a, sem_z)
    d_b = pltpu.async_copy(half.at[pl.ds(s * _WSL + _WSL2, _WSL2)], stage_b,
                           sem_i)
    d_a.wait()
    o_a = pltpu.async_copy(stage_a, out_hbm.at[pl.ds(wbase + s * _WSL, _WSL2)],
                           sem_z)
    d_b.wait()
    o_b = pltpu.async_copy(stage_b,
                           out_hbm.at[pl.ds(wbase + s * _WSL + _WSL2, _WSL2)],
                           sem_i)
    o_a.wait()
    o_b.wait()


def kernel(in_ids, size_tensor):
    assert size_tensor.shape[0] == _MASK and in_ids.shape[0] == _NIDS
    ids = in_ids.astype(jnp.int32)

    mesh = plsc.VectorSubcoreMesh(core_axis_name="c", subcore_axis_name="s",
                                  num_cores=_NCORE, num_subcores=_NSUB)
    run = pl.kernel(
        _scatter_body,
        out_type=jax.ShapeDtypeStruct((_NW,), jnp.int32),
        mesh=mesh,
        compiler_params=pltpu.CompilerParams(needs_layout_passes=False),
        scratch_types=[
            pltpu.VMEM_SHARED((_HW,), jnp.int32),
            pltpu.VMEM((_WIN,), jnp.int32),
            pltpu.VMEM((_N_CH, _CH), jnp.int32),
            pltpu.VMEM((_N_CH, _CH), jnp.int32),
            pltpu.VMEM((_WSL,), jnp.int32),
            pltpu.VMEM((_WSL2,), jnp.int32),
            pltpu.VMEM((_WSL2,), jnp.int32),
            pltpu.SemaphoreType.DMA,
            pltpu.SemaphoreType.DMA,
            pltpu.SemaphoreType.DMA,
        ],
    )
    w = run(ids)
    planes = [((w >> (8 * p)) & 0xFF) != 0 for p in range(4)]
    planes[3] = planes[3][: _MASK - 3 * _NW]
    return jnp.concatenate(planes)
